# baseline (speedup 1.0000x reference)
import jax
import jax.numpy as jnp
from jax.experimental import pallas as pl
from jax.experimental.pallas import tpu as pltpu

SEQ_LEN = 2048
D_MODEL = 1024
BATCH = 4
BS = 512


def _kern(x_ref, t_ref, o_ref):
    i = pl.program_id(0)
    t = t_ref[pl.ds(i * BS, BS), :]
    norm = jnp.sqrt(jnp.sum(t * t, axis=1, keepdims=True))
    scale = jnp.where(norm > 1.0, 1.0 / (norm + 1e-7), 1.0)
    emb = t * scale
    for b in range(BATCH):
        o_ref[:, b, :] = x_ref[:, b, :] + emb


def kernel(x, table):
    return pl.pallas_call(
        _kern,
        grid=(SEQ_LEN // BS,),
        in_specs=[
            pl.BlockSpec((BS, BATCH, D_MODEL), lambda i: (i, 0, 0)),
            pl.BlockSpec((SEQ_LEN, D_MODEL), lambda i: (0, 0)),
        ],
        out_specs=pl.BlockSpec((BS, BATCH, D_MODEL), lambda i: (i, 0, 0)),
        out_shape=jax.ShapeDtypeStruct((SEQ_LEN, BATCH, D_MODEL), x.dtype),
        compiler_params=pltpu.CompilerParams(
            dimension_semantics=("arbitrary",),
        ),
    )(x, table)


# speedup vs baseline: 1.0202x; 1.0202x over previous
import jax
import jax.numpy as jnp
from jax.experimental import pallas as pl
from jax.experimental.pallas import tpu as pltpu

SEQ_LEN = 2048
D_MODEL = 1024
BATCH = 4
BS = 512


def _kern(x_ref, t_ref, o_ref):
    t = t_ref[...]
    norm = jnp.sqrt(jnp.sum(t * t, axis=1, keepdims=True))
    scale = jnp.where(norm > 1.0, 1.0 / (norm + 1e-7), 1.0)
    emb = t * scale
    for b in range(BATCH):
        o_ref[:, b, :] = x_ref[:, b, :] + emb


def kernel(x, table):
    return pl.pallas_call(
        _kern,
        grid=(SEQ_LEN // BS,),
        in_specs=[
            pl.BlockSpec((BS, BATCH, D_MODEL), lambda i: (i, 0, 0)),
            pl.BlockSpec((BS, D_MODEL), lambda i: (i, 0)),
        ],
        out_specs=pl.BlockSpec((BS, BATCH, D_MODEL), lambda i: (i, 0, 0)),
        out_shape=jax.ShapeDtypeStruct((SEQ_LEN, BATCH, D_MODEL), x.dtype),
        compiler_params=pltpu.CompilerParams(
            dimension_semantics=("parallel",),
        ),
    )(x, table)
